# baseline (speedup 1.0000x reference)
import functools
import math
import jax
import jax.numpy as jnp
from jax import lax
from jax.experimental import pallas as pl
from jax.experimental.pallas import tpu as pltpu
from jax.experimental.pallas import tpu_sc as plsc

_T = 8
_TH = tuple(math.log((k / 7) / (1 - k / 7)) for k in range(1, 7)) + (25 * math.log(2),)

_B, _D1, _D2 = 2, 2048, 1024
_NC, _NS = 2, 16
_NW = _NC * _NS
_RPW = (_B * _D1) // _NW
_CR, _CC = 8, 512
_NCH = (_RPW // _CR) * (_D2 // _CC)


_NB = 3


def _sc_body(x_hbm, out_hbm, in_bufs, out_bufs, sin, sout):
    wid = lax.axis_index("s") * _NC + lax.axis_index("c")
    row0 = wid * _RPW
    b = row0 // _D1
    d10 = row0 % _D1

    def in_copy(c, buf, sem):
        d1 = d10 + (c // 2) * _CR
        col = (c % 2) * _CC
        return pltpu.make_async_copy(
            x_hbm.at[b, pl.ds(d1, _CR), pl.ds(col, _CC)], buf, sem)

    def out_copy(c, buf, ti, sem):
        d1 = d10 + (c // 2) * _CR
        col = (c % 2) * _CC
        return pltpu.make_async_copy(
            buf.at[ti], out_hbm.at[b, ti, pl.ds(d1, _CR), pl.ds(col, _CC)], sem)

    def compute(in_v, out_v):
        for r in range(_CR):
            @plsc.parallel_loop(0, _CC, step=16, unroll=4)
            def grp(cc):
                xv = in_v[r, pl.ds(cc, 16)]
                s = [jnp.where(xv >= jnp.float32(th), jnp.float32(1.0),
                               jnp.float32(0.0)) for th in _TH]
                out_v[0, r, pl.ds(cc, 16)] = jnp.float32(1.0) - s[0]
                for k in range(1, _T - 1):
                    out_v[k, r, pl.ds(cc, 16)] = s[k - 1] - s[k]
                out_v[_T - 1, r, pl.ds(cc, 16)] = s[_T - 2]

    def slot(i, c, in_v, out_v, sin_s, sout_s):
        in_copy(c, in_v, sin_s).wait()

        @pl.when(i > 0)
        def _drain_prev():
            for ti in range(_T):
                out_copy(c - _NB, out_v, ti, sout_s).wait()

        compute(in_v, out_v)
        for ti in range(_T):
            out_copy(c, out_v, ti, sout_s).start()

        @pl.when(c + _NB < _NCH)
        def _prefetch_next():
            in_copy(c + _NB, in_v, sin_s).start()

    for j in range(_NB):
        in_copy(j, in_bufs[j], sin[j]).start()

    def it(i, carry):
        for j in range(_NB):
            slot(i, _NB * i + j, in_bufs[j], out_bufs[j], sin[j], sout[j])
        return carry

    lax.fori_loop(0, _NCH // _NB, it, 0)

    rem = _NCH % _NB
    base = _NCH - rem
    for j in range(rem):
        c = base + j
        in_copy(c, in_bufs[j], sin[j]).wait()
        for ti in range(_T):
            out_copy(c - _NB, out_bufs[j], ti, sout[j]).wait()
        compute(in_bufs[j], out_bufs[j])
        for ti in range(_T):
            out_copy(c, out_bufs[j], ti, sout[j]).start()

    for c in range(_NCH - _NB, _NCH):
        j = c % _NB
        for ti in range(_T):
            out_copy(c, out_bufs[j], ti, sout[j]).wait()


@functools.partial(
    pl.kernel,
    mesh=plsc.VectorSubcoreMesh(core_axis_name="c", subcore_axis_name="s"),
    out_type=jax.ShapeDtypeStruct((_B, _T, _D1, _D2), jnp.float32),
    compiler_params=pltpu.CompilerParams(use_tc_tiling_on_sc=True),
    scratch_types=[
        [pltpu.VMEM((_CR, _CC), jnp.float32) for _ in range(_NB)],
        [pltpu.VMEM((_T, _CR, _CC), jnp.float32) for _ in range(_NB)],
        [pltpu.SemaphoreType.DMA for _ in range(_NB)],
        [pltpu.SemaphoreType.DMA for _ in range(_NB)],
    ],
)
def _sc_encode(x_hbm, out_hbm, in_bufs, out_bufs, sin, sout):
    _sc_body(x_hbm, out_hbm, in_bufs, out_bufs, sin, sout)


def kernel(x):
    return _sc_encode(x)
